# baseline (speedup 1.0000x reference)
import functools

import jax
import jax.numpy as jnp
from jax import lax
from jax.experimental import pallas as pl
from jax.experimental.pallas import tpu as pltpu
from jax.experimental.pallas import tpu_sc as plsc

_NUM_CORES = 2
_NUM_SUBCORES = 16
_NUM_WORKERS = _NUM_CORES * _NUM_SUBCORES


def _sc_gather(table, idx, rows, dim):
    b_per_w = rows // _NUM_WORKERS
    mesh = plsc.VectorSubcoreMesh(core_axis_name="c", subcore_axis_name="s")

    @functools.partial(
        pl.kernel,
        mesh=mesh,
        out_type=jax.ShapeDtypeStruct((rows, dim), jnp.float32),
        scratch_types=[
            pltpu.VMEM((b_per_w,), jnp.int32),
            pltpu.VMEM((b_per_w, dim), jnp.float32),
            pltpu.SemaphoreType.DMA,
        ],
    )
    def gather_kernel(table_hbm, idx_hbm, out_hbm, idx_v, rows_v, sem):
        wid = lax.axis_index("s") * _NUM_CORES + lax.axis_index("c")
        base = wid * b_per_w
        pltpu.sync_copy(idx_hbm.at[pl.ds(base, b_per_w)], idx_v)
        pltpu.async_copy(table_hbm.at[idx_v], rows_v, sem).wait()
        pltpu.sync_copy(rows_v, out_hbm.at[pl.ds(base, b_per_w)])

    return gather_kernel(table, idx)


def _add_ln_body(emb_ref, pos_ref, scale_ref, bias_ref, out_ref):
    h = emb_ref[...] + pos_ref[...]
    mean = jnp.mean(h, axis=1, keepdims=True)
    c = h - mean
    var = jnp.mean(c * c, axis=1, keepdims=True)
    normed = c * lax.rsqrt(var + 1e-5) * scale_ref[...] + bias_ref[...]
    out_ref[...] = normed.astype(jnp.bfloat16)


def _matmul_body(var_ref, w_ref, out_ref):
    w = w_ref[...].astype(jnp.bfloat16)
    out_ref[...] = lax.dot_general(
        var_ref[...], w, (((1,), (1,)), ((), ())),
        preferred_element_type=jnp.float32)


def kernel(x, tok_table, pos_table, ln_scale, ln_bias, W_out):
    batch, seq = x.shape
    vocab, emb_dim = tok_table.shape
    rows = batch * seq

    idx = x.reshape(rows).astype(jnp.int32)
    emb = _sc_gather(tok_table, idx, rows, emb_dim)

    pos = jnp.broadcast_to(pos_table[None, :seq, :], (batch, seq, emb_dim))
    pos = pos.reshape(rows, emb_dim)

    var_bf16 = pl.pallas_call(
        _add_ln_body,
        out_shape=jax.ShapeDtypeStruct((rows, emb_dim), jnp.bfloat16),
        in_specs=[
            pl.BlockSpec((rows, emb_dim), lambda: (0, 0)),
            pl.BlockSpec((rows, emb_dim), lambda: (0, 0)),
            pl.BlockSpec((1, emb_dim), lambda: (0, 0)),
            pl.BlockSpec((1, emb_dim), lambda: (0, 0)),
        ],
        out_specs=pl.BlockSpec((rows, emb_dim), lambda: (0, 0)),
    )(emb, pos, ln_scale.reshape(1, emb_dim), ln_bias.reshape(1, emb_dim))

    tile_n = 1792
    grid = (pl.cdiv(vocab, tile_n),)
    logits = pl.pallas_call(
        _matmul_body,
        grid=grid,
        out_shape=jax.ShapeDtypeStruct((rows, vocab), jnp.float32),
        in_specs=[
            pl.BlockSpec((rows, emb_dim), lambda i: (0, 0)),
            pl.BlockSpec((tile_n, emb_dim), lambda i: (i, 0)),
        ],
        out_specs=pl.BlockSpec((rows, tile_n), lambda i: (0, i)),
        compiler_params=pltpu.CompilerParams(
            dimension_semantics=("parallel",),
        ),
    )(var_bf16, W_out)

    return logits.reshape(batch, seq, vocab)


# speedup vs baseline: 1.0051x; 1.0051x over previous
import functools

import jax
import jax.numpy as jnp
from jax import lax
from jax.experimental import pallas as pl
from jax.experimental.pallas import tpu as pltpu
from jax.experimental.pallas import tpu_sc as plsc

_NUM_CORES = 2
_NUM_SUBCORES = 16
_NUM_WORKERS = _NUM_CORES * _NUM_SUBCORES


def _sc_gather(table, idx, rows, dim):
    b_per_w = rows // _NUM_WORKERS
    mesh = plsc.VectorSubcoreMesh(core_axis_name="c", subcore_axis_name="s")

    @functools.partial(
        pl.kernel,
        mesh=mesh,
        out_type=jax.ShapeDtypeStruct((rows, dim), jnp.float32),
        scratch_types=[
            pltpu.VMEM((b_per_w,), jnp.int32),
            pltpu.VMEM((b_per_w, dim), jnp.float32),
            pltpu.SemaphoreType.DMA,
        ],
    )
    def gather_kernel(table_hbm, idx_hbm, out_hbm, idx_v, rows_v, sem):
        wid = lax.axis_index("s") * _NUM_CORES + lax.axis_index("c")
        base = wid * b_per_w
        pltpu.sync_copy(idx_hbm.at[pl.ds(base, b_per_w)], idx_v)
        pltpu.async_copy(table_hbm.at[idx_v], rows_v, sem).wait()
        pltpu.sync_copy(rows_v, out_hbm.at[pl.ds(base, b_per_w)])

    return gather_kernel(table, idx)


def _add_ln_body(emb_ref, pos_ref, scale_ref, bias_ref, out_ref):
    h = emb_ref[...] + pos_ref[...]
    mean = jnp.mean(h, axis=1, keepdims=True)
    c = h - mean
    var = jnp.mean(c * c, axis=1, keepdims=True)
    normed = c * lax.rsqrt(var + 1e-5) * scale_ref[...] + bias_ref[...]
    out_ref[...] = normed.astype(jnp.bfloat16)


def _matmul_body(var_ref, w_ref, out_ref):
    w = w_ref[...].astype(jnp.bfloat16)
    out_ref[...] = lax.dot_general(
        var_ref[...], w, (((1,), (1,)), ((), ())),
        preferred_element_type=jnp.float32)


def kernel(x, tok_table, pos_table, ln_scale, ln_bias, W_out):
    batch, seq = x.shape
    vocab, emb_dim = tok_table.shape
    rows = batch * seq

    idx = x.reshape(rows).astype(jnp.int32)
    emb = _sc_gather(tok_table, idx, rows, emb_dim)

    pos = jnp.broadcast_to(pos_table[None, :seq, :], (batch, seq, emb_dim))
    pos = pos.reshape(rows, emb_dim)

    var_bf16 = pl.pallas_call(
        _add_ln_body,
        out_shape=jax.ShapeDtypeStruct((rows, emb_dim), jnp.bfloat16),
        in_specs=[
            pl.BlockSpec((rows, emb_dim), lambda: (0, 0)),
            pl.BlockSpec((rows, emb_dim), lambda: (0, 0)),
            pl.BlockSpec((1, emb_dim), lambda: (0, 0)),
            pl.BlockSpec((1, emb_dim), lambda: (0, 0)),
        ],
        out_specs=pl.BlockSpec((rows, emb_dim), lambda: (0, 0)),
    )(emb, pos, ln_scale.reshape(1, emb_dim), ln_bias.reshape(1, emb_dim))

    tile_n = 1536
    grid = (pl.cdiv(vocab, tile_n),)
    logits = pl.pallas_call(
        _matmul_body,
        grid=grid,
        out_shape=jax.ShapeDtypeStruct((rows, vocab), jnp.float32),
        in_specs=[
            pl.BlockSpec((rows, emb_dim), lambda i: (0, 0)),
            pl.BlockSpec((tile_n, emb_dim), lambda i: (i, 0)),
        ],
        out_specs=pl.BlockSpec((rows, tile_n), lambda i: (0, i)),
        compiler_params=pltpu.CompilerParams(
            dimension_semantics=("parallel",),
        ),
    )(var_bf16, W_out)

    return logits.reshape(batch, seq, vocab)
